# baseline (speedup 1.0000x reference)
import jax
import jax.numpy as jnp
from jax import lax
from jax.experimental import pallas as pl
from jax.experimental.pallas import tpu as pltpu
from jax.experimental.pallas import tpu_sc as plsc

N = 320000
D = 128
S = 10000
S_PAD = 10240
NC = 2
NS = 16
NW1 = NC * NS
CH = N // NW1
NW2 = NS
SLICE = S_PAD // NW2
L = 16

BLK = 32000


def _mv_body(x_ref, w_in_ref, b_in_ref, w_out_ref, b_out_ref, o_ref):
    w = jnp.dot(w_out_ref[...], w_in_ref[...],
                preferred_element_type=jnp.float32)
    c = jnp.sum(w_out_ref[...] * b_in_ref[...]) + b_out_ref[0, 0]
    o_ref[...] = (jnp.sum(x_ref[...] * w, axis=1) + c).reshape(1, BLK // 128, 128)


def _matvec(x, w_in, b_in, w_out, b_out):
    return pl.pallas_call(
        _mv_body,
        grid=(N // BLK,),
        in_specs=[
            pl.BlockSpec((BLK, D), lambda i: (i, 0)),
            pl.BlockSpec((D, D), lambda i: (0, 0)),
            pl.BlockSpec((1, D), lambda i: (0, 0)),
            pl.BlockSpec((1, D), lambda i: (0, 0)),
            pl.BlockSpec((1, 1), lambda i: (0, 0)),
        ],
        out_specs=pl.BlockSpec((1, BLK // 128, 128), lambda i: (i, 0, 0)),
        out_shape=jax.ShapeDtypeStruct((N // BLK, BLK // 128, 128), jnp.float32),
    )(x, w_in, b_in, w_out, b_out)


_GATHER_DNUMS = lax.GatherDimensionNumbers(
    offset_dims=(), collapsed_slice_dims=(0,), start_index_map=(0,))


def _g16(src, idx):
    return lax.gather(src, idx[:, None], _GATHER_DNUMS, slice_sizes=(1,),
                      mode=lax.GatherScatterMode.PROMISE_IN_BOUNDS)


def _sc1_body(pred_hbm, samples_hbm, sums_out, pres_out,
              s_buf, v_buf, acc, pres, tmp2f, tmp2i, asl, psl,
              sums_sh, pres_sh):
    cid = lax.axis_index("c")
    sid = lax.axis_index("s")
    wid = cid * NS + sid
    base = wid * CH
    iota = lax.iota(jnp.int32, L)
    zeros_f = jnp.zeros((L,), jnp.float32)
    zeros_i = jnp.zeros((L,), jnp.int32)
    ones_i = jnp.ones((L,), jnp.int32)

    def zero_body(j, _):
        acc[pl.ds(j * L, L)] = zeros_f
        pres[pl.ds(j * L, L)] = zeros_i
        return 0
    lax.fori_loop(0, S_PAD // L, zero_body, 0)

    pltpu.sync_copy(pred_hbm.at[pl.ds(base, CH)], v_buf)
    pltpu.sync_copy(samples_hbm.at[pl.ds(base, CH)], s_buf.at[pl.ds(0, CH)])

    @plsc.parallel_loop(0, CH // L, unroll=16)
    def scan_body(i):
        s_cur = s_buf[pl.ds(i * L, L)]
        s_nx = s_buf[pl.ds(i * L + 1, L)]
        v = v_buf[pl.ds(i * L, L)]
        ends14 = (s_cur != s_nx) & (iota < 15)
        c = plsc.cumsum(v)
        m = plsc.cummax(jnp.where(ends14, iota, -1))
        prev = jnp.where(iota >= 1, _g16(m, jnp.maximum(iota - 1, 0)), -1)
        base_c = jnp.where(prev >= 0, _g16(c, jnp.maximum(prev, 0)), 0.0)
        val = c - base_c
        mask = ends14 | (iota == 15)
        plsc.addupdate_scatter(acc, [s_cur], val, mask=mask)
        plsc.addupdate_scatter(pres, [s_cur], ones_i, mask=mask)

    pltpu.sync_copy(acc, sums_sh.at[sid])
    pltpu.sync_copy(pres, pres_sh.at[sid])
    plsc.subcore_barrier()

    off = sid * SLICE
    pltpu.sync_copy(sums_sh.at[:, pl.ds(off, SLICE)], tmp2f)
    pltpu.sync_copy(pres_sh.at[:, pl.ds(off, SLICE)], tmp2i)

    def bzero(j, _):
        asl[pl.ds(j * L, L)] = zeros_f
        psl[pl.ds(j * L, L)] = zeros_i
        return 0
    lax.fori_loop(0, SLICE // L, bzero, 0)

    def red_tile(t, _):
        def add_vec(j, _):
            asl[pl.ds(j * L, L)] = asl[pl.ds(j * L, L)] + tmp2f[t, pl.ds(j * L, L)]
            psl[pl.ds(j * L, L)] = psl[pl.ds(j * L, L)] + tmp2i[t, pl.ds(j * L, L)]
            return 0
        lax.fori_loop(0, SLICE // L, add_vec, 0)
        return 0
    lax.fori_loop(0, NS, red_tile, 0)

    pltpu.sync_copy(asl, sums_out.at[cid, pl.ds(off, SLICE)])
    pltpu.sync_copy(psl, pres_out.at[cid, pl.ds(off, SLICE)])


def _sc2_body(sums_hbm, pres_hbm, out_hbm,
              t2f, t2i, accv, presv, out_v):
    wid = lax.axis_index("s")
    zeros_f = jnp.zeros((L,), jnp.float32)
    zeros_i = jnp.zeros((L,), jnp.int32)
    full15 = jnp.full((L,), 15, jnp.int32)

    @pl.when(wid == 0)
    def _():
        pltpu.sync_copy(sums_hbm, t2f)
        pltpu.sync_copy(pres_hbm, t2i)

        def comb(j, cnt):
            sl = pl.ds(j * L, L)
            accv[sl] = t2f[0, sl] + t2f[1, sl]
            p = t2i[0, sl] + t2i[1, sl]
            presv[sl] = p
            return cnt + (p > 0).astype(jnp.int32)
        cnt = lax.fori_loop(0, S_PAD // L, comb, zeros_i)
        n_present = jnp.sum(cnt)

        @pl.when(n_present == S)
        def _():
            pltpu.sync_copy(accv.at[pl.ds(0, S)], out_hbm)

        @pl.when(n_present != S)
        def _():
            def czero(j, _):
                out_v[pl.ds(j * L, L)] = zeros_f
                return 0
            lax.fori_loop(0, S_PAD // L, czero, 0)

            def rank_body(j, run):
                p = presv[pl.ds(j * L, L)] > 0
                p01 = p.astype(jnp.int32)
                incl = plsc.cumsum(p01)
                rank = run + incl - p01
                sv = accv[pl.ds(j * L, L)]
                plsc.store_scatter(out_v, [rank], sv, mask=p)
                return run + _g16(incl, full15)

            lax.fori_loop(0, S_PAD // L, rank_body, zeros_i)
            pltpu.sync_copy(out_v.at[pl.ds(0, S)], out_hbm)


def _sc_segsum(pred, samples):
    mesh1 = plsc.VectorSubcoreMesh(core_axis_name="c", subcore_axis_name="s",
                                   num_cores=NC, num_subcores=NS)
    sums, pres = pl.kernel(
        _sc1_body,
        out_type=(jax.ShapeDtypeStruct((NC, S_PAD), jnp.float32),
                  jax.ShapeDtypeStruct((NC, S_PAD), jnp.int32)),
        mesh=mesh1,
        compiler_params=pltpu.CompilerParams(needs_layout_passes=False),
        scratch_types=[
            pltpu.VMEM((CH + L,), jnp.int32),
            pltpu.VMEM((CH,), jnp.float32),
            pltpu.VMEM((S_PAD,), jnp.float32),
            pltpu.VMEM((S_PAD,), jnp.int32),
            pltpu.VMEM((NS, SLICE), jnp.float32),
            pltpu.VMEM((NS, SLICE), jnp.int32),
            pltpu.VMEM((SLICE,), jnp.float32),
            pltpu.VMEM((SLICE,), jnp.int32),
            pltpu.VMEM_SHARED((NS, S_PAD), jnp.float32),
            pltpu.VMEM_SHARED((NS, S_PAD), jnp.int32),
        ],
    )(pred, samples)

    mesh2 = plsc.VectorSubcoreMesh(core_axis_name="c", subcore_axis_name="s",
                                   num_cores=1, num_subcores=NS)
    return pl.kernel(
        _sc2_body,
        out_type=jax.ShapeDtypeStruct((S,), jnp.float32),
        mesh=mesh2,
        compiler_params=pltpu.CompilerParams(needs_layout_passes=False),
        scratch_types=[
            pltpu.VMEM((NC, S_PAD), jnp.float32),
            pltpu.VMEM((NC, S_PAD), jnp.int32),
            pltpu.VMEM((S_PAD,), jnp.float32),
            pltpu.VMEM((S_PAD,), jnp.int32),
            pltpu.VMEM((S_PAD,), jnp.float32),
        ],
    )(sums, pres)


def kernel(x, samples, W_in, b_in, W_out, b_out):
    pred = _matvec(x, W_in, b_in.reshape(1, D), W_out, b_out.reshape(1, 1))
    out = _sc_segsum(pred.reshape(N), samples)
    return out.reshape(S, 1)


# speedup vs baseline: 1.0232x; 1.0232x over previous
import jax
import jax.numpy as jnp
from jax import lax
from jax.experimental import pallas as pl
from jax.experimental.pallas import tpu as pltpu
from jax.experimental.pallas import tpu_sc as plsc

N = 320000
D = 128
S = 10000
S_PAD = 10240
NC = 2
NS = 16
NW1 = NC * NS
CH = N // NW1
NW2 = NS
SLICE = S_PAD // NW2
L = 16

BLK = 32000


def _mv_body(x_ref, w_in_ref, b_in_ref, w_out_ref, b_out_ref, o_ref):
    w = jnp.dot(w_out_ref[...], w_in_ref[...],
                preferred_element_type=jnp.float32)
    c = jnp.sum(w_out_ref[...] * b_in_ref[...]) + b_out_ref[0, 0]
    o_ref[...] = (jnp.sum(x_ref[...] * w, axis=1) + c).reshape(1, BLK // 128, 128)


def _matvec(x, w_in, b_in, w_out, b_out):
    return pl.pallas_call(
        _mv_body,
        grid=(N // BLK,),
        in_specs=[
            pl.BlockSpec((BLK, D), lambda i: (i, 0)),
            pl.BlockSpec((D, D), lambda i: (0, 0)),
            pl.BlockSpec((1, D), lambda i: (0, 0)),
            pl.BlockSpec((1, D), lambda i: (0, 0)),
            pl.BlockSpec((1, 1), lambda i: (0, 0)),
        ],
        out_specs=pl.BlockSpec((1, BLK // 128, 128), lambda i: (i, 0, 0)),
        out_shape=jax.ShapeDtypeStruct((N // BLK, BLK // 128, 128), jnp.float32),
    )(x, w_in, b_in, w_out, b_out)


_GATHER_DNUMS = lax.GatherDimensionNumbers(
    offset_dims=(), collapsed_slice_dims=(0,), start_index_map=(0,))


def _g16(src, idx):
    return lax.gather(src, idx[:, None], _GATHER_DNUMS, slice_sizes=(1,),
                      mode=lax.GatherScatterMode.PROMISE_IN_BOUNDS)


def _sc1_body(pred_hbm, samples_hbm, sums_out, pres_out,
              s_buf, v_buf, acc, pres, tmp2f, tmp2i, asl, psl,
              sums_sh, pres_sh):
    cid = lax.axis_index("c")
    sid = lax.axis_index("s")
    wid = cid * NS + sid
    base = wid * CH
    iota = lax.iota(jnp.int32, L)
    zeros_f = jnp.zeros((L,), jnp.float32)
    zeros_i = jnp.zeros((L,), jnp.int32)
    ones_i = jnp.ones((L,), jnp.int32)

    def zero_body(j, _):
        acc[pl.ds(j * L, L)] = zeros_f
        pres[pl.ds(j * L, L)] = zeros_i
        return 0
    lax.fori_loop(0, S_PAD // L, zero_body, 0)

    pltpu.sync_copy(pred_hbm.at[pl.ds(base, CH)], v_buf)
    pltpu.sync_copy(samples_hbm.at[pl.ds(base, CH)], s_buf.at[pl.ds(0, CH)])

    @plsc.parallel_loop(0, CH // L, unroll=4)
    def scan_body(i):
        s_cur = s_buf[pl.ds(i * L, L)]
        s_nx = s_buf[pl.ds(i * L + 1, L)]
        v = v_buf[pl.ds(i * L, L)]
        ends14 = (s_cur != s_nx) & (iota < 15)
        c = plsc.cumsum(v)
        m = plsc.cummax(jnp.where(ends14, iota, -1))
        prev = jnp.where(iota >= 1, _g16(m, jnp.maximum(iota - 1, 0)), -1)
        base_c = jnp.where(prev >= 0, _g16(c, jnp.maximum(prev, 0)), 0.0)
        val = c - base_c
        mask = ends14 | (iota == 15)
        plsc.addupdate_scatter(acc, [s_cur], val, mask=mask)
        plsc.addupdate_scatter(pres, [s_cur], ones_i, mask=mask)

    pltpu.sync_copy(acc, sums_sh.at[sid])
    pltpu.sync_copy(pres, pres_sh.at[sid])
    plsc.subcore_barrier()

    off = sid * SLICE
    pltpu.sync_copy(sums_sh.at[:, pl.ds(off, SLICE)], tmp2f)
    pltpu.sync_copy(pres_sh.at[:, pl.ds(off, SLICE)], tmp2i)

    def bzero(j, _):
        asl[pl.ds(j * L, L)] = zeros_f
        psl[pl.ds(j * L, L)] = zeros_i
        return 0
    lax.fori_loop(0, SLICE // L, bzero, 0)

    def red_tile(t, _):
        def add_vec(j, _):
            asl[pl.ds(j * L, L)] = asl[pl.ds(j * L, L)] + tmp2f[t, pl.ds(j * L, L)]
            psl[pl.ds(j * L, L)] = psl[pl.ds(j * L, L)] + tmp2i[t, pl.ds(j * L, L)]
            return 0
        lax.fori_loop(0, SLICE // L, add_vec, 0)
        return 0
    lax.fori_loop(0, NS, red_tile, 0)

    pltpu.sync_copy(asl, sums_out.at[cid, pl.ds(off, SLICE)])
    pltpu.sync_copy(psl, pres_out.at[cid, pl.ds(off, SLICE)])


def _sc2_body(sums_hbm, pres_hbm, out_hbm,
              t2f, t2i, accv, presv, out_v):
    wid = lax.axis_index("s")
    zeros_f = jnp.zeros((L,), jnp.float32)
    zeros_i = jnp.zeros((L,), jnp.int32)
    full15 = jnp.full((L,), 15, jnp.int32)

    @pl.when(wid == 0)
    def _():
        pltpu.sync_copy(sums_hbm, t2f)
        pltpu.sync_copy(pres_hbm, t2i)

        def comb(j, cnt):
            sl = pl.ds(j * L, L)
            accv[sl] = t2f[0, sl] + t2f[1, sl]
            p = t2i[0, sl] + t2i[1, sl]
            presv[sl] = p
            return cnt + (p > 0).astype(jnp.int32)
        cnt = lax.fori_loop(0, S_PAD // L, comb, zeros_i)
        n_present = jnp.sum(cnt)

        @pl.when(n_present == S)
        def _():
            pltpu.sync_copy(accv.at[pl.ds(0, S)], out_hbm)

        @pl.when(n_present != S)
        def _():
            def czero(j, _):
                out_v[pl.ds(j * L, L)] = zeros_f
                return 0
            lax.fori_loop(0, S_PAD // L, czero, 0)

            def rank_body(j, run):
                p = presv[pl.ds(j * L, L)] > 0
                p01 = p.astype(jnp.int32)
                incl = plsc.cumsum(p01)
                rank = run + incl - p01
                sv = accv[pl.ds(j * L, L)]
                plsc.store_scatter(out_v, [rank], sv, mask=p)
                return run + _g16(incl, full15)

            lax.fori_loop(0, S_PAD // L, rank_body, zeros_i)
            pltpu.sync_copy(out_v.at[pl.ds(0, S)], out_hbm)


def _sc_segsum(pred, samples):
    mesh1 = plsc.VectorSubcoreMesh(core_axis_name="c", subcore_axis_name="s",
                                   num_cores=NC, num_subcores=NS)
    sums, pres = pl.kernel(
        _sc1_body,
        out_type=(jax.ShapeDtypeStruct((NC, S_PAD), jnp.float32),
                  jax.ShapeDtypeStruct((NC, S_PAD), jnp.int32)),
        mesh=mesh1,
        compiler_params=pltpu.CompilerParams(needs_layout_passes=False),
        scratch_types=[
            pltpu.VMEM((CH + L,), jnp.int32),
            pltpu.VMEM((CH,), jnp.float32),
            pltpu.VMEM((S_PAD,), jnp.float32),
            pltpu.VMEM((S_PAD,), jnp.int32),
            pltpu.VMEM((NS, SLICE), jnp.float32),
            pltpu.VMEM((NS, SLICE), jnp.int32),
            pltpu.VMEM((SLICE,), jnp.float32),
            pltpu.VMEM((SLICE,), jnp.int32),
            pltpu.VMEM_SHARED((NS, S_PAD), jnp.float32),
            pltpu.VMEM_SHARED((NS, S_PAD), jnp.int32),
        ],
    )(pred, samples)

    mesh2 = plsc.VectorSubcoreMesh(core_axis_name="c", subcore_axis_name="s",
                                   num_cores=1, num_subcores=NS)
    return pl.kernel(
        _sc2_body,
        out_type=jax.ShapeDtypeStruct((S,), jnp.float32),
        mesh=mesh2,
        compiler_params=pltpu.CompilerParams(needs_layout_passes=False),
        scratch_types=[
            pltpu.VMEM((NC, S_PAD), jnp.float32),
            pltpu.VMEM((NC, S_PAD), jnp.int32),
            pltpu.VMEM((S_PAD,), jnp.float32),
            pltpu.VMEM((S_PAD,), jnp.int32),
            pltpu.VMEM((S_PAD,), jnp.float32),
        ],
    )(sums, pres)


def kernel(x, samples, W_in, b_in, W_out, b_out):
    pred = _matvec(x, W_in, b_in.reshape(1, D), W_out, b_out.reshape(1, 1))
    out = _sc_segsum(pred.reshape(N), samples)
    return out.reshape(S, 1)
